# Initial kernel scaffold; baseline (speedup 1.0000x reference)
import jax
import jax.numpy as jnp
from jax.experimental import pallas as pl


def kernel(user_ids, item_ids, user_table, item_table, W1u, b1u, W2u, b2u, W1i, b1i, W2i, b2i):
    raise NotImplementedError("write your pallas kernel here")


# speedup vs baseline: 8.4523x; 8.4523x over previous
import functools

import jax
import jax.numpy as jnp
from jax import lax
from jax.experimental import pallas as pl
from jax.experimental.pallas import tpu as pltpu
from jax.experimental.pallas import tpu_sc as plsc

BATCH = 16384
D = 128
NC = 2
NS = 16
NW = NC * NS
BPW = BATCH // NW
CHUNK = 128
NCH = BPW // CHUNK


def _gather_body(user_table, item_table, uidx_hbm, iidx_hbm, u_out, v_out,
                 uidx_v, iidx_v, rows_v, sem):
    wid = lax.axis_index("s") * NC + lax.axis_index("c")
    base = wid * BPW
    pltpu.sync_copy(uidx_hbm.at[wid], uidx_v)
    pltpu.sync_copy(iidx_hbm.at[wid], iidx_v)
    cps = [
        pltpu.async_copy(user_table.at[uidx_v.at[c]],
                         rows_v.at[pl.ds(c * CHUNK, CHUNK)], sem)
        for c in range(NCH)
    ]
    for cp in cps:
        cp.wait()
    pltpu.sync_copy(rows_v, u_out.at[pl.ds(base, BPW)])
    cps = [
        pltpu.async_copy(item_table.at[iidx_v.at[c]],
                         rows_v.at[pl.ds(c * CHUNK, CHUNK)], sem)
        for c in range(NCH)
    ]
    for cp in cps:
        cp.wait()
    pltpu.sync_copy(rows_v, v_out.at[pl.ds(base, BPW)])


@functools.lru_cache(maxsize=1)
def _make_gather():
    mesh = plsc.VectorSubcoreMesh(core_axis_name="c", subcore_axis_name="s")
    return functools.partial(
        pl.kernel,
        mesh=mesh,
        out_type=[
            jax.ShapeDtypeStruct((BATCH, D), jnp.float32),
            jax.ShapeDtypeStruct((BATCH, D), jnp.float32),
        ],
        scratch_types=[
            pltpu.VMEM((NCH, CHUNK), jnp.int32),
            pltpu.VMEM((NCH, CHUNK), jnp.int32),
            pltpu.VMEM((BPW, D), jnp.float32),
            pltpu.SemaphoreType.DMA,
        ],
    )(_gather_body)


BLK = 1024


def _mlp_body(uv, iv, w1u, b1u, w2u, b2u, w1i, b1i, w2i, b2i, u_out, v_out):
    def tower(x, w1, b1, w2, b2):
        h = lax.dot_general(x, w1, (((1,), (1,)), ((), ())),
                            preferred_element_type=jnp.float32)
        h = jnp.maximum(h + b1, 0.0)
        y = lax.dot_general(h, w2, (((1,), (1,)), ((), ())),
                            preferred_element_type=jnp.float32) + b2
        n = jnp.sqrt(jnp.sum(y * y, axis=1, keepdims=True))
        return y / jnp.maximum(n, 1e-12)

    u_out[...] = tower(uv[...], w1u[...], b1u[...], w2u[...], b2u[...])
    v_out[...] = tower(iv[...], w1i[...], b1i[...], w2i[...], b2i[...])


def _mlp(u_vecs, v_vecs, W1u, b1u, W2u, b2u, W1i, b1i, W2i, b2i):
    vec_spec = pl.BlockSpec((BLK, D), lambda i: (i, 0))
    w_spec = pl.BlockSpec((D, D), lambda i: (0, 0))
    b_spec = pl.BlockSpec((1, D), lambda i: (0, 0))
    return pl.pallas_call(
        _mlp_body,
        grid=(BATCH // BLK,),
        in_specs=[vec_spec, vec_spec,
                  w_spec, b_spec, w_spec, b_spec,
                  w_spec, b_spec, w_spec, b_spec],
        out_specs=[vec_spec, vec_spec],
        out_shape=[
            jax.ShapeDtypeStruct((BATCH, D), jnp.float32),
            jax.ShapeDtypeStruct((BATCH, D), jnp.float32),
        ],
    )(u_vecs, v_vecs, W1u, b1u.reshape(1, D), W2u, b2u.reshape(1, D),
      W1i, b1i.reshape(1, D), W2i, b2i.reshape(1, D))


def kernel(user_ids, item_ids, user_table, item_table,
           W1u, b1u, W2u, b2u, W1i, b1i, W2i, b2i):
    uidx = user_ids.astype(jnp.int32).reshape(NW, NCH, CHUNK)
    iidx = item_ids.astype(jnp.int32).reshape(NW, NCH, CHUNK)
    u_vecs, v_vecs = _make_gather()(user_table, item_table, uidx, iidx)
    u, v = _mlp(u_vecs, v_vecs, W1u, b1u, W2u, b2u, W1i, b1i, W2i, b2i)
    return (u, v)
